# baseline (speedup 1.0000x reference)
import numpy as np
import jax
import jax.numpy as jnp
from jax.experimental import pallas as pl
from jax.experimental.pallas import tpu as pltpu

_SPLIT_PROB = 0.05
_MERGE_PROB = 0.2
_DISTURB_PROB = 0.4
_DISTURB_ALPHA = 0.2
_EPS = 1e-12
_BSZ, _NSPK, _DIM, _T = 16, 16, 256, 2048
_LANES = 128
_ROWS = (_T * _NSPK) // _LANES


def _build_plan():
    rng = np.random.default_rng(0)
    spk_count = np.zeros(_NSPK, np.float32)
    spk_count[: _NSPK - 4] = 1.0
    norm = np.ones(_NSPK, np.float32)
    mask = np.ones((_BSZ, _NSPK), np.float32)
    ops = []
    prob = rng.random(_BSZ)
    for idx in np.nonzero(prob < _DISTURB_PROB)[0]:
        pos = np.nonzero(spk_count * mask[idx])[0]
        valid = np.nonzero(norm * mask[idx])[0]
        if len(pos) == 0 or len(valid) == 0:
            continue
        kt = int(rng.integers(len(pos)))
        kd = int(rng.integers(len(valid)))
        alpha = _DISTURB_ALPHA * float(rng.random())
        mask[idx, pos[kt]] = 0
        ops.append(("disturb", int(idx), kt, kd, alpha, None))
    prob = rng.random(_BSZ)
    for idx in np.nonzero(prob < _SPLIT_PROB)[0]:
        valid = np.nonzero(spk_count * mask[idx])[0]
        pad = np.nonzero((spk_count == 0) * mask[idx])[0]
        if len(valid) == 0 or len(pad) == 0:
            continue
        ks = int(rng.integers(len(valid)))
        kc = int(rng.integers(len(pad)))
        dvec = rng.standard_normal(_DIM).astype(np.float32)
        dvec = dvec / max(np.linalg.norm(dvec), _EPS)
        mask[idx, valid[ks]] = 0
        mask[idx, pad[kc]] = 0
        ops.append(("split", int(idx), ks, kc, None, dvec))
    prob = rng.random(_BSZ)
    for idx in np.nonzero(prob < _MERGE_PROB)[0]:
        valid = np.nonzero(norm * mask[idx])[0]
        if len(valid) == 0:
            continue
        k1 = int(rng.integers(len(valid)))
        k2 = int(rng.integers(len(valid)))
        mask[idx, valid[k1]] = 0
        mask[idx, valid[k2]] = 0
        ops.append(("merge", int(idx), k1, k2, None, None))
    per_batch = [[] for _ in range(_BSZ)]
    for op in ops:
        per_batch[op[1]].append(op)
    return per_batch


_PER_BATCH = _build_plan()


_N_DMA = 8


def _body(sp_ref, prof_ref, bl_ref, sp_out, prof_out, bl_out, sems):
    nb = _BSZ // _N_DMA
    copies = [
        pltpu.make_async_copy(
            sp_ref.at[pl.ds(i * nb, nb)], sp_out.at[pl.ds(i * nb, nb)],
            sems.at[i])
        for i in range(_N_DMA)
    ]
    for c in copies:
        c.start()
    _replay_body(prof_ref, bl_ref, prof_out, bl_out)
    for c in copies:
        c.wait()


def _replay_body(prof_ref, bl_ref, prof_out, bl_out):
    f32 = jnp.float32
    lane16 = jax.lax.broadcasted_iota(jnp.int32, (1, _NSPK), 1)
    ii = jax.lax.broadcasted_iota(jnp.int32, (_NSPK, _NSPK), 0)
    jj = jax.lax.broadcasted_iota(jnp.int32, (_NSPK, _NSPK), 1)
    tri = (ii <= jj).astype(f32)
    li = jax.lax.broadcasted_iota(jnp.int32, (_LANES, _NSPK), 0)
    si = jax.lax.broadcasted_iota(jnp.int32, (_LANES, _NSPK), 1)
    fold = ((li % _NSPK) == si).astype(f32)
    bi = jax.lax.broadcasted_iota(jnp.int32, (_LANES, _LANES), 0)
    bj = jax.lax.broadcasted_iota(jnp.int32, (_LANES, _LANES), 1)
    bdiag = ((bi // _NSPK) == (bj // _NSPK)).astype(f32)
    lmod = jax.lax.broadcasted_iota(jnp.int32, (_ROWS, _LANES), 1) % _NSPK
    row_ids = jax.lax.broadcasted_iota(jnp.int32, (_NSPK, 1), 0)

    def kth_nonzero(nzrow, k):
        cs = jnp.dot(nzrow, tri, preferred_element_type=f32)
        eq = cs == float(k + 1)
        first = jnp.min(jnp.where(eq, lane16, _NSPK))
        return jnp.where(first == _NSPK, 0, first)

    def norms_row(p):
        return jax.lax.dot_general(
            jnp.ones((1, _DIM), f32), p * p,
            dimension_numbers=(((1,), (1,)), ((), ())),
            preferred_element_type=f32)

    def get_row(p, a):
        sel = (lane16 == a).astype(f32)
        return jnp.dot(sel, p, preferred_element_type=f32)

    def set_row(p, a, v):
        return jnp.where(row_ids == a, v, p)

    def nrm(v):
        n = jnp.sqrt(jnp.sum(v * v))
        return v / jnp.maximum(n, _EPS)

    for b in range(_BSZ):
        ops = _PER_BATCH[b]
        pb = prof_ref[b]
        n2 = jnp.sum(pb * pb, axis=1, keepdims=True)
        pb = pb / jnp.maximum(jnp.sqrt(n2), _EPS)

        if not ops:
            prof_out[b] = pb
            bl_out[b] = bl_ref[b]
            continue

        maskv = jnp.ones((1, _NSPK), f32)
        needs_spk = any(op[0] in ("disturb", "split") for op in ops)
        if needs_spk:
            colsum = jnp.sum(bl_ref[b], axis=0).reshape(1, _LANES)
            spk = jnp.dot(colsum, fold, preferred_element_type=f32)
            spk_nz = (spk != 0.0).astype(f32)

        has_merge = any(op[0] == "merge" for op in ops)
        zb = bl_ref[b] if has_merge else None

        for kind, _, ka, kb, alpha, dvec in ops:
            if kind == "disturb":
                nz = spk_nz * (maskv != 0.0).astype(f32)
                a = kth_nonzero(nz, ka)
                nrm2 = norms_row(pb)
                nzn = ((nrm2 != 0.0) & (maskv != 0.0)).astype(f32)
                d = kth_nonzero(nzn, kb)
                v = (1.0 - alpha) * get_row(pb, a) + alpha * get_row(pb, d)
                pb = set_row(pb, a, nrm(v))
                maskv = jnp.where(lane16 == a, 0.0, maskv)
            elif kind == "split":
                nz = spk_nz * (maskv != 0.0).astype(f32)
                a = kth_nonzero(nz, ka)
                nzp = ((spk == 0.0) & (maskv != 0.0)).astype(f32)
                c = kth_nonzero(nzp, kb)
                v = get_row(pb, a) + _DISTURB_ALPHA * jnp.asarray(
                    dvec, f32).reshape(1, _DIM)
                pb = set_row(pb, c, nrm(v))
                maskv = jnp.where(lane16 == a, 0.0, maskv)
                maskv = jnp.where(lane16 == c, 0.0, maskv)
            else:
                nrm2 = norms_row(pb)
                nzn = ((nrm2 != 0.0) & (maskv != 0.0)).astype(f32)
                a = kth_nonzero(nzn, ka)
                d = kth_nonzero(nzn, kb)
                v = get_row(pb, a) + get_row(pb, d)
                pb = set_row(pb, a, nrm(v))
                pb = set_row(pb, d, jnp.zeros((1, _DIM), f32))
                sa = lmod == a
                sd = lmod == d
                av = jnp.dot(jnp.where(sa, zb, 0.0), bdiag,
                             preferred_element_type=f32)
                dv = jnp.dot(jnp.where(sd, zb, 0.0), bdiag,
                             preferred_element_type=f32)
                m = ((av + dv) > 0.0).astype(f32)
                zb = jnp.where(sd, 0.0, jnp.where(sa, m, zb))
                maskv = jnp.where(lane16 == a, 0.0, maskv)
                maskv = jnp.where(lane16 == d, 0.0, maskv)

        prof_out[b] = pb
        bl_out[b] = zb if has_merge else bl_ref[b]


def kernel(speech, profile, binary_labels):
    bl = binary_labels.reshape(_BSZ, _ROWS, _LANES)
    sp_out, prof_out, bl_out = pl.pallas_call(
        _body,
        in_specs=[
            pl.BlockSpec(memory_space=pl.ANY),
            pl.BlockSpec(memory_space=pltpu.VMEM),
            pl.BlockSpec(memory_space=pltpu.VMEM),
        ],
        out_specs=[
            pl.BlockSpec(memory_space=pl.ANY),
            pl.BlockSpec(memory_space=pltpu.VMEM),
            pl.BlockSpec(memory_space=pltpu.VMEM),
        ],
        out_shape=[
            jax.ShapeDtypeStruct(speech.shape, jnp.float32),
            jax.ShapeDtypeStruct((_BSZ, _NSPK, _DIM), jnp.float32),
            jax.ShapeDtypeStruct((_BSZ, _ROWS, _LANES), jnp.float32),
        ],
        scratch_shapes=[pltpu.SemaphoreType.DMA((_N_DMA,))],
    )(speech, profile, bl)
    return (sp_out, prof_out, bl_out.reshape(_BSZ, _T, _NSPK))


# speedup vs baseline: 16.3450x; 16.3450x over previous
import functools
import numpy as np
import jax
import jax.numpy as jnp
from jax import lax
from jax.experimental import pallas as pl
from jax.experimental.pallas import tpu as pltpu
from jax.experimental.pallas import tpu_sc as plsc

_SPLIT_PROB = 0.05
_MERGE_PROB = 0.2
_DISTURB_PROB = 0.4
_DISTURB_ALPHA = 0.2
_EPS = 1e-12
_BSZ, _NSPK, _DIM, _T = 16, 16, 256, 2048
_NC, _NS = 2, 16
_HALF = _T // 2
_NCH = _DIM // _NSPK


def _build_plan():
    rng = np.random.default_rng(0)
    spk_count = np.zeros(_NSPK, np.float32)
    spk_count[: _NSPK - 4] = 1.0
    norm = np.ones(_NSPK, np.float32)
    mask = np.ones((_BSZ, _NSPK), np.float32)
    ops = []
    prob = rng.random(_BSZ)
    for idx in np.nonzero(prob < _DISTURB_PROB)[0]:
        pos = np.nonzero(spk_count * mask[idx])[0]
        valid = np.nonzero(norm * mask[idx])[0]
        if len(pos) == 0 or len(valid) == 0:
            continue
        kt = int(rng.integers(len(pos)))
        kd = int(rng.integers(len(valid)))
        alpha = _DISTURB_ALPHA * float(rng.random())
        mask[idx, pos[kt]] = 0
        ops.append(("disturb", int(idx), kt, kd, alpha, None))
    prob = rng.random(_BSZ)
    for idx in np.nonzero(prob < _SPLIT_PROB)[0]:
        valid = np.nonzero(spk_count * mask[idx])[0]
        pad = np.nonzero((spk_count == 0) * mask[idx])[0]
        if len(valid) == 0 or len(pad) == 0:
            continue
        ks = int(rng.integers(len(valid)))
        kc = int(rng.integers(len(pad)))
        dvec = rng.standard_normal(_DIM).astype(np.float32)
        dvec = dvec / max(np.linalg.norm(dvec), _EPS)
        mask[idx, valid[ks]] = 0
        mask[idx, pad[kc]] = 0
        ops.append(("split", int(idx), ks, kc, None, dvec))
    prob = rng.random(_BSZ)
    for idx in np.nonzero(prob < _MERGE_PROB)[0]:
        valid = np.nonzero(norm * mask[idx])[0]
        if len(valid) == 0:
            continue
        k1 = int(rng.integers(len(valid)))
        k2 = int(rng.integers(len(valid)))
        mask[idx, valid[k1]] = 0
        mask[idx, valid[k2]] = 0
        ops.append(("merge", int(idx), k1, k2, None, None))
    per_batch = [[] for _ in range(_BSZ)]
    for op in ops:
        per_batch[op[1]].append(op)
    return per_batch


_PER_BATCH = _build_plan()
_MERGE_BATCHES = [b for b in range(_BSZ)
                  if any(op[0] == "merge" for op in _PER_BATCH[b])]


def _make_sc_call():
    mesh = plsc.VectorSubcoreMesh(core_axis_name="c", subcore_axis_name="s",
                                  num_cores=_NC, num_subcores=_NS)
    f32, i32 = jnp.float32, jnp.int32

    @functools.partial(
        pl.kernel, mesh=mesh,
        compiler_params=pltpu.CompilerParams(needs_layout_passes=False),
        out_type=[
            jax.ShapeDtypeStruct((_BSZ * _NSPK * _DIM,), f32),
            jax.ShapeDtypeStruct((_BSZ * _T * _NSPK,), f32),
        ],
        scratch_types=[
            pltpu.VMEM((_T * _NSPK,), f32),
            pltpu.VMEM((_NSPK * _DIM,), f32),
            pltpu.VMEM((_DIM,), f32),
        ],
    )
    def sc_kernel(prof_hbm, bl_hbm, prof_out, bl_out, chunk, prof_v, row_v):
        c = lax.axis_index("c")
        s = lax.axis_index("s")
        iota = lax.iota(i32, _NSPK)
        zeros = jnp.zeros((_NSPK,), f32)

        def rsqrt_nt(x):
            i = lax.bitcast_convert_type(x, i32)
            y = lax.bitcast_convert_type(
                jnp.full((_NSPK,), 0x5F3759DF, i32) - (i >> 1), f32)
            for _ in range(3):
                y = y * (1.5 - 0.5 * x * y * y)
            return y

        def inv_norm(ssv):
            return jnp.where(ssv >= 1e-24,
                             rsqrt_nt(jnp.maximum(ssv, 1e-24)),
                             jnp.full((_NSPK,), 1.0 / _EPS, f32))

        def kth(nzf, k):
            cs = jnp.cumsum(nzf)
            cnt_f = jnp.sum(jnp.where(cs <= float(k), 1.0, 0.0))
            cnt = jnp.full((_NSPK,), cnt_f, f32).astype(i32)
            return jnp.where((cnt >= _NSPK) | (cnt < 0), 0, cnt)

        for cc in range(_NC):
            for jj in range(8):
                b = cc * 8 + jj

                @pl.when((c == cc) & (s == 2 * jj))
                def _batch(b=b):
                    ops = _PER_BATCH[b]
                    pltpu.sync_copy(
                        bl_hbm.at[pl.ds(b * _T * _NSPK, _T * _NSPK)], chunk)
                    pltpu.sync_copy(
                        prof_hbm.at[pl.ds(b * _NSPK * _DIM, _NSPK * _DIM)],
                        prof_v)

                    if ops:
                        def p1(t, acc):
                            return acc + plsc.load_gather(
                                chunk, [t * _NSPK + iota])

                        spk = lax.fori_loop(0, _T, p1, zeros)
                        spk_nz = spk != 0.0

                    def nbody(sr, norms2):
                        base = sr * _DIM

                        def nin(ch, a2):
                            v = plsc.load_gather(
                                prof_v, [base + _NSPK * ch + iota])
                            return a2 + v * v

                        a2 = lax.fori_loop(0, _NCH, nin, zeros)
                        ssv = jnp.full((_NSPK,), jnp.sum(a2), f32)
                        inv = inv_norm(ssv)

                        def nsc(ch, carry):
                            idxs = base + _NSPK * ch + iota
                            v = plsc.load_gather(prof_v, [idxs])
                            plsc.store_scatter(prof_v, [idxs], v * inv)
                            return carry

                        lax.fori_loop(0, _NCH, nsc, 0)
                        return jnp.where(iota == sr, ssv * inv * inv, norms2)

                    norms2 = lax.fori_loop(0, _NSPK, nbody, zeros)
                    maskv = jnp.ones((_NSPK,), f32)

                    def row_pass(a_vec, d_vec, combine):
                        abase = a_vec * _DIM
                        dbase = d_vec * _DIM if d_vec is not None else None

                        def rb(ch, a2):
                            idx1 = _NSPK * ch + iota
                            ra = plsc.load_gather(prof_v, [abase + idx1])
                            if dbase is not None:
                                rd = plsc.load_gather(prof_v, [dbase + idx1])
                            else:
                                rd = None
                            v = combine(ch, ra, rd)
                            plsc.store_scatter(row_v, [idx1], v)
                            return a2 + v * v

                        a2 = lax.fori_loop(0, _NCH, rb, zeros)
                        return jnp.full((_NSPK,), jnp.sum(a2), f32)

                    def write_row(dst_vec, inv, zero_vec=None):
                        dstb = dst_vec * _DIM
                        zb = zero_vec * _DIM if zero_vec is not None else None

                        def wb(ch, carry):
                            idx1 = _NSPK * ch + iota
                            v = plsc.load_gather(row_v, [idx1])
                            plsc.store_scatter(prof_v, [dstb + idx1], v * inv)
                            if zb is not None:
                                plsc.store_scatter(prof_v, [zb + idx1], zeros)
                            return carry

                        lax.fori_loop(0, _NCH, wb, 0)

                    for kind, _, ka, kb, alpha, dvec in ops:
                        mask_nz = maskv != 0.0
                        if kind == "disturb":
                            a_vec = kth(
                                jnp.where(spk_nz & mask_nz, 1.0, 0.0), ka)
                            d_vec = kth(
                                jnp.where((norms2 != 0.0) & mask_nz,
                                          1.0, 0.0), kb)
                            ssv = row_pass(
                                a_vec, d_vec,
                                lambda ch, ra, rd, al=alpha:
                                    (1.0 - al) * ra + al * rd)
                            inv = inv_norm(ssv)
                            write_row(a_vec, inv)
                            norms2 = jnp.where(iota == a_vec,
                                               ssv * inv * inv, norms2)
                            maskv = jnp.where(iota == a_vec, 0.0, maskv)
                        elif kind == "split":
                            a_vec = kth(
                                jnp.where(spk_nz & mask_nz, 1.0, 0.0), ka)
                            c_vec = kth(
                                jnp.where((~spk_nz) & mask_nz, 1.0, 0.0), kb)
                            dchunks = [
                                jnp.asarray(dvec[ch * _NSPK:(ch + 1) * _NSPK],
                                            f32) for ch in range(_NCH)]

                            def comb(ch, ra, rd, dchunks=dchunks):
                                dsel = dchunks[0] * 0.0
                                for cidx in range(_NCH):
                                    dsel = jnp.where(ch == cidx,
                                                     dchunks[cidx], dsel)
                                return ra + _DISTURB_ALPHA * dsel

                            ssv = row_pass(a_vec, None, comb)
                            inv = inv_norm(ssv)
                            write_row(c_vec, inv)
                            norms2 = jnp.where(iota == c_vec,
                                               ssv * inv * inv, norms2)
                            maskv = jnp.where(iota == a_vec, 0.0, maskv)
                            maskv = jnp.where(iota == c_vec, 0.0, maskv)
                        else:
                            nzn = jnp.where((norms2 != 0.0) & mask_nz,
                                            1.0, 0.0)
                            a_vec = kth(nzn, ka)
                            d_vec = kth(nzn, kb)
                            ssv = row_pass(a_vec, d_vec,
                                           lambda ch, ra, rd: ra + rd)
                            inv = inv_norm(ssv)
                            write_row(a_vec, inv, zero_vec=d_vec)
                            norms2 = jnp.where(iota == a_vec,
                                               ssv * inv * inv, norms2)
                            norms2 = jnp.where(iota == d_vec, 0.0, norms2)
                            maskv = jnp.where(iota == a_vec, 0.0, maskv)
                            maskv = jnp.where(iota == d_vec, 0.0, maskv)

                            def fx(g, carry, a_vec=a_vec, d_vec=d_vec):
                                rows = (iota + _NSPK * g) * _NSPK
                                ca = plsc.load_gather(chunk, [rows + a_vec])
                                cd = plsc.load_gather(chunk, [rows + d_vec])
                                m = jnp.where(ca + cd > 0.0, 1.0, 0.0)
                                plsc.store_scatter(chunk, [rows + a_vec], m)
                                plsc.store_scatter(chunk, [rows + d_vec],
                                                   zeros)
                                return carry

                            lax.fori_loop(0, _T // _NSPK, fx, 0)

                    pltpu.sync_copy(
                        prof_v,
                        prof_out.at[pl.ds(b * _NSPK * _DIM, _NSPK * _DIM)])
                    pltpu.sync_copy(
                        chunk, bl_out.at[pl.ds(b * _T * _NSPK, _T * _NSPK)])

    return sc_kernel


_SC_CALL_CACHE = []


def kernel(speech, profile, binary_labels):
    if not _SC_CALL_CACHE:
        _SC_CALL_CACHE.append(_make_sc_call())
    prof_flat = profile.reshape(_BSZ * _NSPK * _DIM)
    bl_flat = binary_labels.reshape(_BSZ * _T * _NSPK)
    prof_out, bl_out = _SC_CALL_CACHE[0](prof_flat, bl_flat)
    return (speech, prof_out.reshape(_BSZ, _NSPK, _DIM),
            bl_out.reshape(_BSZ, _T, _NSPK))


# speedup vs baseline: 16.9597x; 1.0376x over previous
import functools
import numpy as np
import jax
import jax.numpy as jnp
from jax import lax
from jax.experimental import pallas as pl
from jax.experimental.pallas import tpu as pltpu
from jax.experimental.pallas import tpu_sc as plsc

_SPLIT_PROB = 0.05
_MERGE_PROB = 0.2
_DISTURB_PROB = 0.4
_DISTURB_ALPHA = 0.2
_EPS = 1e-12
_BSZ, _NSPK, _DIM, _T = 16, 16, 256, 2048
_NC, _NS = 2, 16
_HALF = _T // 2
_NCH = _DIM // _NSPK


def _build_plan():
    rng = np.random.default_rng(0)
    spk_count = np.zeros(_NSPK, np.float32)
    spk_count[: _NSPK - 4] = 1.0
    norm = np.ones(_NSPK, np.float32)
    mask = np.ones((_BSZ, _NSPK), np.float32)
    ops = []
    prob = rng.random(_BSZ)
    for idx in np.nonzero(prob < _DISTURB_PROB)[0]:
        pos = np.nonzero(spk_count * mask[idx])[0]
        valid = np.nonzero(norm * mask[idx])[0]
        if len(pos) == 0 or len(valid) == 0:
            continue
        kt = int(rng.integers(len(pos)))
        kd = int(rng.integers(len(valid)))
        alpha = _DISTURB_ALPHA * float(rng.random())
        mask[idx, pos[kt]] = 0
        ops.append(("disturb", int(idx), kt, kd, alpha, None))
    prob = rng.random(_BSZ)
    for idx in np.nonzero(prob < _SPLIT_PROB)[0]:
        valid = np.nonzero(spk_count * mask[idx])[0]
        pad = np.nonzero((spk_count == 0) * mask[idx])[0]
        if len(valid) == 0 or len(pad) == 0:
            continue
        ks = int(rng.integers(len(valid)))
        kc = int(rng.integers(len(pad)))
        dvec = rng.standard_normal(_DIM).astype(np.float32)
        dvec = dvec / max(np.linalg.norm(dvec), _EPS)
        mask[idx, valid[ks]] = 0
        mask[idx, pad[kc]] = 0
        ops.append(("split", int(idx), ks, kc, None, dvec))
    prob = rng.random(_BSZ)
    for idx in np.nonzero(prob < _MERGE_PROB)[0]:
        valid = np.nonzero(norm * mask[idx])[0]
        if len(valid) == 0:
            continue
        k1 = int(rng.integers(len(valid)))
        k2 = int(rng.integers(len(valid)))
        mask[idx, valid[k1]] = 0
        mask[idx, valid[k2]] = 0
        ops.append(("merge", int(idx), k1, k2, None, None))
    per_batch = [[] for _ in range(_BSZ)]
    for op in ops:
        per_batch[op[1]].append(op)
    return per_batch


_PER_BATCH = _build_plan()
_MERGE_BATCHES = [b for b in range(_BSZ)
                  if any(op[0] == "merge" for op in _PER_BATCH[b])]


def _make_sc_call():
    mesh = plsc.VectorSubcoreMesh(core_axis_name="c", subcore_axis_name="s",
                                  num_cores=_NC, num_subcores=_NS)
    f32, i32 = jnp.float32, jnp.int32

    @functools.partial(
        pl.kernel, mesh=mesh,
        compiler_params=pltpu.CompilerParams(needs_layout_passes=False),
        out_type=[
            jax.ShapeDtypeStruct((_BSZ * _NSPK * _DIM,), f32),
            jax.ShapeDtypeStruct((_BSZ * _T * _NSPK,), f32),
        ],
        scratch_types=[
            pltpu.VMEM((_T * _NSPK,), f32),
            pltpu.VMEM((_NSPK * _DIM,), f32),
        ],
    )
    def sc_kernel(prof_hbm, bl_hbm, prof_out, bl_out, chunk, prof_v):
        c = lax.axis_index("c")
        s = lax.axis_index("s")
        iota = lax.iota(i32, _NSPK)
        zeros = jnp.zeros((_NSPK,), f32)

        def rsqrt_nt(x):
            i = lax.bitcast_convert_type(x, i32)
            y = lax.bitcast_convert_type(
                jnp.full((_NSPK,), 0x5F3759DF, i32) - (i >> 1), f32)
            for _ in range(3):
                y = y * (1.5 - 0.5 * x * y * y)
            return y

        def inv_norm(ssv):
            return jnp.where(ssv >= 1e-24,
                             rsqrt_nt(jnp.maximum(ssv, 1e-24)),
                             jnp.full((_NSPK,), 1.0 / _EPS, f32))

        def kth(nzf, k):
            cs = jnp.cumsum(nzf)
            cnt_f = jnp.sum(jnp.where(cs <= float(k), 1.0, 0.0))
            cnt = jnp.full((_NSPK,), cnt_f, f32).astype(i32)
            return jnp.where((cnt >= _NSPK) | (cnt < 0), 0, cnt)

        def load_row(base_vec):
            return [plsc.load_gather(prof_v, [base_vec + _NSPK * ch + iota])
                    for ch in range(_NCH)]

        def sumsq(vs):
            a0 = zeros
            a1 = zeros
            for ch, v in enumerate(vs):
                if ch % 2 == 0:
                    a0 = a0 + v * v
                else:
                    a1 = a1 + v * v
            return jnp.full((_NSPK,), jnp.sum(a0 + a1), f32)

        for cc in range(_NC):
            for jj in range(8):
                b = cc * 8 + jj

                @pl.when((c == cc) & (s == 2 * jj))
                def _batch(b=b):
                    ops = _PER_BATCH[b]
                    pltpu.sync_copy(
                        bl_hbm.at[pl.ds(b * _T * _NSPK, _T * _NSPK)], chunk)
                    pltpu.sync_copy(
                        prof_hbm.at[pl.ds(b * _NSPK * _DIM, _NSPK * _DIM)],
                        prof_v)

                    if any(op[0] in ("disturb", "split") for op in ops):
                        def p1(t, accs):
                            a0, a1, a2, a3 = accs
                            base = t * (_NSPK * 16)
                            for u in range(4):
                                o = base + 4 * u * _NSPK
                                a0 = a0 + plsc.load_gather(chunk, [o + iota])
                                a1 = a1 + plsc.load_gather(
                                    chunk, [o + _NSPK + iota])
                                a2 = a2 + plsc.load_gather(
                                    chunk, [o + 2 * _NSPK + iota])
                                a3 = a3 + plsc.load_gather(
                                    chunk, [o + 3 * _NSPK + iota])
                            return (a0, a1, a2, a3)

                        accs = lax.fori_loop(0, _T // 16, p1,
                                             (zeros, zeros, zeros, zeros))
                        spk = accs[0] + accs[1] + accs[2] + accs[3]
                        spk_nz = spk != 0.0

                    def nbody(sr, norms2):
                        base = sr * _DIM
                        vs = load_row(base)
                        ssv = sumsq(vs)
                        inv = inv_norm(ssv)
                        for ch, v in enumerate(vs):
                            plsc.store_scatter(
                                prof_v, [base + _NSPK * ch + iota], v * inv)
                        return jnp.where(iota == sr, ssv * inv * inv, norms2)

                    norms2 = lax.fori_loop(0, _NSPK, nbody, zeros)
                    maskv = jnp.ones((_NSPK,), f32)

                    def write_row(vs, dst_vec, inv, zero_vec=None):
                        dstb = dst_vec * _DIM
                        zb = zero_vec * _DIM if zero_vec is not None else None
                        for ch, v in enumerate(vs):
                            idx1 = _NSPK * ch + iota
                            plsc.store_scatter(prof_v, [dstb + idx1], v * inv)
                            if zb is not None:
                                plsc.store_scatter(prof_v, [zb + idx1], zeros)

                    for kind, _, ka, kb, alpha, dvec in ops:
                        mask_nz = maskv != 0.0
                        if kind == "disturb":
                            a_vec = kth(
                                jnp.where(spk_nz & mask_nz, 1.0, 0.0), ka)
                            d_vec = kth(
                                jnp.where((norms2 != 0.0) & mask_nz,
                                          1.0, 0.0), kb)
                            ras = load_row(a_vec * _DIM)
                            rds = load_row(d_vec * _DIM)
                            vs = [(1.0 - alpha) * ra + alpha * rd
                                  for ra, rd in zip(ras, rds)]
                            ssv = sumsq(vs)
                            inv = inv_norm(ssv)
                            write_row(vs, a_vec, inv)
                            norms2 = jnp.where(iota == a_vec,
                                               ssv * inv * inv, norms2)
                            maskv = jnp.where(iota == a_vec, 0.0, maskv)
                        elif kind == "split":
                            a_vec = kth(
                                jnp.where(spk_nz & mask_nz, 1.0, 0.0), ka)
                            c_vec = kth(
                                jnp.where((~spk_nz) & mask_nz, 1.0, 0.0), kb)
                            ras = load_row(a_vec * _DIM)
                            vs = [ra + _DISTURB_ALPHA * jnp.asarray(
                                      dvec[ch * _NSPK:(ch + 1) * _NSPK], f32)
                                  for ch, ra in enumerate(ras)]
                            ssv = sumsq(vs)
                            inv = inv_norm(ssv)
                            write_row(vs, c_vec, inv)
                            norms2 = jnp.where(iota == c_vec,
                                               ssv * inv * inv, norms2)
                            maskv = jnp.where(iota == a_vec, 0.0, maskv)
                            maskv = jnp.where(iota == c_vec, 0.0, maskv)
                        else:
                            nzn = jnp.where((norms2 != 0.0) & mask_nz,
                                            1.0, 0.0)
                            a_vec = kth(nzn, ka)
                            d_vec = kth(nzn, kb)
                            ras = load_row(a_vec * _DIM)
                            rds = load_row(d_vec * _DIM)
                            vs = [ra + rd for ra, rd in zip(ras, rds)]
                            ssv = sumsq(vs)
                            inv = inv_norm(ssv)
                            write_row(vs, a_vec, inv, zero_vec=d_vec)
                            norms2 = jnp.where(iota == a_vec,
                                               ssv * inv * inv, norms2)
                            norms2 = jnp.where(iota == d_vec, 0.0, norms2)
                            maskv = jnp.where(iota == a_vec, 0.0, maskv)
                            maskv = jnp.where(iota == d_vec, 0.0, maskv)

                            def fx(g, carry, a_vec=a_vec, d_vec=d_vec):
                                for u in range(4):
                                    rows = (iota + _NSPK * (4 * g + u)) * _NSPK
                                    ca = plsc.load_gather(
                                        chunk, [rows + a_vec])
                                    cd = plsc.load_gather(
                                        chunk, [rows + d_vec])
                                    m = jnp.where(ca + cd > 0.0, 1.0, 0.0)
                                    plsc.store_scatter(
                                        chunk, [rows + a_vec], m)
                                    plsc.store_scatter(
                                        chunk, [rows + d_vec], zeros)
                                return carry

                            lax.fori_loop(0, _T // _NSPK // 4, fx, 0)

                    pltpu.sync_copy(
                        prof_v,
                        prof_out.at[pl.ds(b * _NSPK * _DIM, _NSPK * _DIM)])
                    pltpu.sync_copy(
                        chunk, bl_out.at[pl.ds(b * _T * _NSPK, _T * _NSPK)])

    return sc_kernel


_SC_CALL_CACHE = []


def kernel(speech, profile, binary_labels):
    if not _SC_CALL_CACHE:
        _SC_CALL_CACHE.append(_make_sc_call())
    prof_flat = profile.reshape(_BSZ * _NSPK * _DIM)
    bl_flat = binary_labels.reshape(_BSZ * _T * _NSPK)
    prof_out, bl_out = _SC_CALL_CACHE[0](prof_flat, bl_flat)
    return (speech, prof_out.reshape(_BSZ, _NSPK, _DIM),
            bl_out.reshape(_BSZ, _T, _NSPK))


# speedup vs baseline: 17.2993x; 1.0200x over previous
import functools
import numpy as np
import jax
import jax.numpy as jnp
from jax import lax
from jax.experimental import pallas as pl
from jax.experimental.pallas import tpu as pltpu
from jax.experimental.pallas import tpu_sc as plsc

_SPLIT_PROB = 0.05
_MERGE_PROB = 0.2
_DISTURB_PROB = 0.4
_DISTURB_ALPHA = 0.2
_EPS = 1e-12
_BSZ, _NSPK, _DIM, _T = 16, 16, 256, 2048
_NC, _NS = 2, 16
_HALF = _T // 2
_NCH = _DIM // _NSPK


def _build_plan():
    rng = np.random.default_rng(0)
    spk_count = np.zeros(_NSPK, np.float32)
    spk_count[: _NSPK - 4] = 1.0
    norm = np.ones(_NSPK, np.float32)
    mask = np.ones((_BSZ, _NSPK), np.float32)
    ops = []
    prob = rng.random(_BSZ)
    for idx in np.nonzero(prob < _DISTURB_PROB)[0]:
        pos = np.nonzero(spk_count * mask[idx])[0]
        valid = np.nonzero(norm * mask[idx])[0]
        if len(pos) == 0 or len(valid) == 0:
            continue
        kt = int(rng.integers(len(pos)))
        kd = int(rng.integers(len(valid)))
        alpha = _DISTURB_ALPHA * float(rng.random())
        mask[idx, pos[kt]] = 0
        ops.append(("disturb", int(idx), kt, kd, alpha, None))
    prob = rng.random(_BSZ)
    for idx in np.nonzero(prob < _SPLIT_PROB)[0]:
        valid = np.nonzero(spk_count * mask[idx])[0]
        pad = np.nonzero((spk_count == 0) * mask[idx])[0]
        if len(valid) == 0 or len(pad) == 0:
            continue
        ks = int(rng.integers(len(valid)))
        kc = int(rng.integers(len(pad)))
        dvec = rng.standard_normal(_DIM).astype(np.float32)
        dvec = dvec / max(np.linalg.norm(dvec), _EPS)
        mask[idx, valid[ks]] = 0
        mask[idx, pad[kc]] = 0
        ops.append(("split", int(idx), ks, kc, None, dvec))
    prob = rng.random(_BSZ)
    for idx in np.nonzero(prob < _MERGE_PROB)[0]:
        valid = np.nonzero(norm * mask[idx])[0]
        if len(valid) == 0:
            continue
        k1 = int(rng.integers(len(valid)))
        k2 = int(rng.integers(len(valid)))
        mask[idx, valid[k1]] = 0
        mask[idx, valid[k2]] = 0
        ops.append(("merge", int(idx), k1, k2, None, None))
    per_batch = [[] for _ in range(_BSZ)]
    for op in ops:
        per_batch[op[1]].append(op)
    return per_batch


_PER_BATCH = _build_plan()
_MERGE_BATCHES = [b for b in range(_BSZ)
                  if any(op[0] == "merge" for op in _PER_BATCH[b])]


def _make_sc_call():
    mesh = plsc.VectorSubcoreMesh(core_axis_name="c", subcore_axis_name="s",
                                  num_cores=1, num_subcores=_NS)
    f32, i32 = jnp.float32, jnp.int32

    @functools.partial(
        pl.kernel, mesh=mesh,
        compiler_params=pltpu.CompilerParams(needs_layout_passes=False),
        out_type=[
            jax.ShapeDtypeStruct((_BSZ * _NSPK * _DIM,), f32),
            jax.ShapeDtypeStruct((_BSZ * _T * _NSPK,), f32),
        ],
        scratch_types=[
            pltpu.VMEM((_T * _NSPK,), f32),
            pltpu.VMEM((_NSPK * _DIM,), f32),
        ],
    )
    def sc_kernel(prof_hbm, bl_hbm, prof_out, bl_out, chunk, prof_v):
        c = lax.axis_index("c")
        s = lax.axis_index("s")
        iota = lax.iota(i32, _NSPK)
        zeros = jnp.zeros((_NSPK,), f32)

        def rsqrt_nt(x):
            i = lax.bitcast_convert_type(x, i32)
            y = lax.bitcast_convert_type(
                jnp.full((_NSPK,), 0x5F3759DF, i32) - (i >> 1), f32)
            for _ in range(3):
                y = y * (1.5 - 0.5 * x * y * y)
            return y

        def inv_norm(ssv):
            return jnp.where(ssv >= 1e-24,
                             rsqrt_nt(jnp.maximum(ssv, 1e-24)),
                             jnp.full((_NSPK,), 1.0 / _EPS, f32))

        def kth(nzf, k):
            cs = jnp.cumsum(nzf)
            cnt_f = jnp.sum(jnp.where(cs <= float(k), 1.0, 0.0))
            cnt = jnp.full((_NSPK,), cnt_f, f32).astype(i32)
            return jnp.where((cnt >= _NSPK) | (cnt < 0), 0, cnt)

        def load_row(base_vec):
            return [plsc.load_gather(prof_v, [base_vec + _NSPK * ch + iota])
                    for ch in range(_NCH)]

        def sumsq(vs):
            a0 = zeros
            a1 = zeros
            for ch, v in enumerate(vs):
                if ch % 2 == 0:
                    a0 = a0 + v * v
                else:
                    a1 = a1 + v * v
            return jnp.full((_NSPK,), jnp.sum(a0 + a1), f32)

        for b in range(_BSZ):

                @pl.when(s == b)
                def _batch(b=b):
                    ops = _PER_BATCH[b]
                    pltpu.sync_copy(
                        bl_hbm.at[pl.ds(b * _T * _NSPK, _T * _NSPK)], chunk)
                    pltpu.sync_copy(
                        prof_hbm.at[pl.ds(b * _NSPK * _DIM, _NSPK * _DIM)],
                        prof_v)

                    if any(op[0] in ("disturb", "split") for op in ops):
                        def p1(t, accs):
                            a0, a1, a2, a3 = accs
                            base = t * (_NSPK * 16)
                            for u in range(4):
                                o = base + 4 * u * _NSPK
                                a0 = a0 + chunk[pl.ds(o, _NSPK)]
                                a1 = a1 + chunk[pl.ds(o + _NSPK, _NSPK)]
                                a2 = a2 + chunk[pl.ds(o + 2 * _NSPK, _NSPK)]
                                a3 = a3 + chunk[pl.ds(o + 3 * _NSPK, _NSPK)]
                            return (a0, a1, a2, a3)

                        accs = lax.fori_loop(0, _T // 16, p1,
                                             (zeros, zeros, zeros, zeros))
                        spk = accs[0] + accs[1] + accs[2] + accs[3]
                        spk_nz = spk != 0.0

                    def nbody(sr, norms2):
                        base = sr * _DIM
                        vs = load_row(base)
                        ssv = sumsq(vs)
                        inv = inv_norm(ssv)
                        for ch, v in enumerate(vs):
                            plsc.store_scatter(
                                prof_v, [base + _NSPK * ch + iota], v * inv)
                        return jnp.where(iota == sr, ssv * inv * inv, norms2)

                    norms2 = lax.fori_loop(0, _NSPK, nbody, zeros)
                    maskv = jnp.ones((_NSPK,), f32)

                    def write_row(vs, dst_vec, inv, zero_vec=None):
                        dstb = dst_vec * _DIM
                        zb = zero_vec * _DIM if zero_vec is not None else None
                        for ch, v in enumerate(vs):
                            idx1 = _NSPK * ch + iota
                            plsc.store_scatter(prof_v, [dstb + idx1], v * inv)
                            if zb is not None:
                                plsc.store_scatter(prof_v, [zb + idx1], zeros)

                    for kind, _, ka, kb, alpha, dvec in ops:
                        mask_nz = maskv != 0.0
                        if kind == "disturb":
                            a_vec = kth(
                                jnp.where(spk_nz & mask_nz, 1.0, 0.0), ka)
                            d_vec = kth(
                                jnp.where((norms2 != 0.0) & mask_nz,
                                          1.0, 0.0), kb)
                            ras = load_row(a_vec * _DIM)
                            rds = load_row(d_vec * _DIM)
                            vs = [(1.0 - alpha) * ra + alpha * rd
                                  for ra, rd in zip(ras, rds)]
                            ssv = sumsq(vs)
                            inv = inv_norm(ssv)
                            write_row(vs, a_vec, inv)
                            norms2 = jnp.where(iota == a_vec,
                                               ssv * inv * inv, norms2)
                            maskv = jnp.where(iota == a_vec, 0.0, maskv)
                        elif kind == "split":
                            a_vec = kth(
                                jnp.where(spk_nz & mask_nz, 1.0, 0.0), ka)
                            c_vec = kth(
                                jnp.where((~spk_nz) & mask_nz, 1.0, 0.0), kb)
                            ras = load_row(a_vec * _DIM)
                            vs = [ra + _DISTURB_ALPHA * jnp.asarray(
                                      dvec[ch * _NSPK:(ch + 1) * _NSPK], f32)
                                  for ch, ra in enumerate(ras)]
                            ssv = sumsq(vs)
                            inv = inv_norm(ssv)
                            write_row(vs, c_vec, inv)
                            norms2 = jnp.where(iota == c_vec,
                                               ssv * inv * inv, norms2)
                            maskv = jnp.where(iota == a_vec, 0.0, maskv)
                            maskv = jnp.where(iota == c_vec, 0.0, maskv)
                        else:
                            nzn = jnp.where((norms2 != 0.0) & mask_nz,
                                            1.0, 0.0)
                            a_vec = kth(nzn, ka)
                            d_vec = kth(nzn, kb)
                            ras = load_row(a_vec * _DIM)
                            rds = load_row(d_vec * _DIM)
                            vs = [ra + rd for ra, rd in zip(ras, rds)]
                            ssv = sumsq(vs)
                            inv = inv_norm(ssv)
                            write_row(vs, a_vec, inv, zero_vec=d_vec)
                            norms2 = jnp.where(iota == a_vec,
                                               ssv * inv * inv, norms2)
                            norms2 = jnp.where(iota == d_vec, 0.0, norms2)
                            maskv = jnp.where(iota == a_vec, 0.0, maskv)
                            maskv = jnp.where(iota == d_vec, 0.0, maskv)

                            def fx(g, carry, a_vec=a_vec, d_vec=d_vec):
                                for u in range(4):
                                    rows = (iota + _NSPK * (4 * g + u)) * _NSPK
                                    ca = plsc.load_gather(
                                        chunk, [rows + a_vec])
                                    cd = plsc.load_gather(
                                        chunk, [rows + d_vec])
                                    m = jnp.where(ca + cd > 0.0, 1.0, 0.0)
                                    plsc.store_scatter(
                                        chunk, [rows + a_vec], m)
                                    plsc.store_scatter(
                                        chunk, [rows + d_vec], zeros)
                                return carry

                            lax.fori_loop(0, _T // _NSPK // 4, fx, 0)

                    pltpu.sync_copy(
                        prof_v,
                        prof_out.at[pl.ds(b * _NSPK * _DIM, _NSPK * _DIM)])
                    pltpu.sync_copy(
                        chunk, bl_out.at[pl.ds(b * _T * _NSPK, _T * _NSPK)])

    return sc_kernel


_SC_CALL_CACHE = []


def kernel(speech, profile, binary_labels):
    if not _SC_CALL_CACHE:
        _SC_CALL_CACHE.append(_make_sc_call())
    prof_flat = profile.reshape(_BSZ * _NSPK * _DIM)
    bl_flat = binary_labels.reshape(_BSZ * _T * _NSPK)
    prof_out, bl_out = _SC_CALL_CACHE[0](prof_flat, bl_flat)
    return (speech, prof_out.reshape(_BSZ, _NSPK, _DIM),
            bl_out.reshape(_BSZ, _T, _NSPK))
